# baseline (speedup 1.0000x reference)
import functools

import jax
import jax.numpy as jnp
from jax import lax
from jax.experimental import pallas as pl
from jax.experimental.pallas import tpu as pltpu
from jax.experimental.pallas import tpu_sc as plsc

_PREC = lax.Precision.HIGHEST

_NC = 2
_NS = 16
_NW = _NC * _NS


def _lrelu(v):
    return jnp.where(v > 0, v, 0.01 * v)


def _dot(a, b):
    return jnp.dot(a, b, preferred_element_type=jnp.float32, precision=_PREC)


def _bdot(a, b):
    return jnp.dot(a.astype(jnp.bfloat16), b.astype(jnp.bfloat16),
                   preferred_element_type=jnp.float32)


def _sc_gather(table, idx, chunk=2000):
    E = idx.shape[0]
    D = table.shape[1]
    per_w = E // _NW
    mesh = plsc.VectorSubcoreMesh(core_axis_name="c", subcore_axis_name="s")

    @functools.partial(
        pl.kernel,
        out_type=jax.ShapeDtypeStruct((E, D), jnp.float32),
        mesh=mesh,
        scratch_types=[
            pltpu.VMEM((chunk,), jnp.int32),
            pltpu.VMEM((chunk, D), jnp.float32),
            pltpu.SemaphoreType.DMA,
        ],
        compiler_params=pltpu.CompilerParams(use_tc_tiling_on_sc=False),
    )
    def k(table_hbm, idx_hbm, out_hbm, idx_v, rows_v, sem):
        wid = lax.axis_index("s") * _NC + lax.axis_index("c")
        base = wid * per_w

        @pl.loop(0, per_w, step=chunk)
        def _(off):
            pltpu.sync_copy(idx_hbm.at[pl.ds(base + off, chunk)], idx_v)
            pltpu.async_copy(table_hbm.at[idx_v], rows_v, sem).wait()
            pltpu.sync_copy(rows_v, out_hbm.at[pl.ds(base + off, chunk)])

    return k(table, idx)


def _sc_scatter_add(vals, idx, zeros, chunk=2000):
    E, D = vals.shape
    n_rows = zeros.shape[0]
    per_w = E // _NW
    rows_per_s = n_rows // _NS
    mesh = plsc.VectorSubcoreMesh(core_axis_name="c", subcore_axis_name="s")

    @functools.partial(
        pl.kernel,
        out_type=jax.ShapeDtypeStruct((_NC, n_rows, D), jnp.float32),
        mesh=mesh,
        scratch_types=[
            pltpu.VMEM((chunk,), jnp.int32),
            pltpu.VMEM((chunk, D), jnp.float32),
            pltpu.VMEM_SHARED((n_rows, D), jnp.float32),
        ],
        compiler_params=pltpu.CompilerParams(use_tc_tiling_on_sc=False),
    )
    def k(vals_hbm, idx_hbm, zeros_hbm, out_hbm, idx_v, val_v, acc_sh):
        cid = lax.axis_index("c")
        sid = lax.axis_index("s")
        wid = sid * _NC + cid
        row0 = sid * rows_per_s
        pltpu.sync_copy(zeros_hbm.at[pl.ds(row0, rows_per_s)],
                        acc_sh.at[pl.ds(row0, rows_per_s)])
        plsc.subcore_barrier()

        base = wid * per_w

        @pl.loop(0, per_w, step=chunk)
        def _(off):
            pltpu.sync_copy(idx_hbm.at[pl.ds(base + off, chunk)], idx_v)
            pltpu.sync_copy(vals_hbm.at[pl.ds(base + off, chunk)], val_v)
            pltpu.sync_copy(val_v, acc_sh.at[idx_v], add=True)

        plsc.subcore_barrier()
        pltpu.sync_copy(acc_sh.at[pl.ds(row0, rows_per_s)],
                        out_hbm.at[cid, pl.ds(row0, rows_per_s)])

    return k(vals, idx, zeros)


def _full(a):
    return pl.BlockSpec(a.shape, lambda i: (0,) * a.ndim)


def _node_encoder_tc(x, tw, Wn1, Bn1, Wn2, Bn2, Wt, Bt, block=2000):
    N = x.shape[0]
    DIM = Wn1.shape[1]

    def body(x_ref, tw_ref, wn1, bn1, wn2, bn2, wt, bt, out_ref):
        h1 = _lrelu(_bdot(x_ref[...], wn1[...]) + bn1[...])
        h = _bdot(h1, wn2[...]) + bn2[...]
        t = tw_ref[...] * wt[...] + bt[...]
        out_ref[...] = h * t

    return pl.pallas_call(
        body,
        grid=(N // block,),
        in_specs=[
            pl.BlockSpec((block, x.shape[1]), lambda i: (i, 0)),
            pl.BlockSpec((block, 1), lambda i: (i, 0)),
            _full(Wn1), _full(Bn1), _full(Wn2), _full(Bn2),
            _full(Wt), _full(Bt),
        ],
        out_specs=pl.BlockSpec((block, DIM), lambda i: (i, 0)),
        out_shape=jax.ShapeDtypeStruct((N, DIM), jnp.float32),
    )(x, tw, Wn1, Bn1, Wn2, Bn2, Wt, Bt)


def _edge_mlp_tc(hs, er, We1, Be1, We2, Be2, w1a, w1b, b1, w2, b2, w3, b3,
                 block=8000):
    E = hs.shape[0]
    Dh = hs.shape[1]
    De = er.shape[1]
    Dout = w3.shape[1]
    grid = (E // block,)

    def body(hs_ref, er_ref, we1, be1, we2, be2,
             w1a_r, w1b_r, b1_r, w2_r, b2_r, w3_r, b3_r, out_ref):
        ea = _bdot(_lrelu(_bdot(er_ref[...], we1[...]) + be1[...]),
                   we2[...]) + be2[...]
        mm = _lrelu(_bdot(hs_ref[...], w1a_r[...]) +
                    _bdot(ea, w1b_r[...]) + b1_r[...])
        mm = _lrelu(_bdot(mm, w2_r[...]) + b2_r[...])
        out_ref[...] = _bdot(mm, w3_r[...]) + b3_r[...]

    return pl.pallas_call(
        body,
        grid=grid,
        in_specs=[
            pl.BlockSpec((block, Dh), lambda i: (i, 0)),
            pl.BlockSpec((block, De), lambda i: (i, 0)),
            _full(We1), _full(Be1), _full(We2), _full(Be2),
            _full(w1a), _full(w1b), _full(b1), _full(w2), _full(b2),
            _full(w3), _full(b3),
        ],
        out_specs=pl.BlockSpec((block, Dout), lambda i: (i, 0)),
        out_shape=jax.ShapeDtypeStruct((E, Dout), jnp.float32),
    )(hs, er, We1, Be1, We2, Be2, w1a, w1b, b1, w2, b2, w3, b3)


def _combine_bn_tc(agg, h, root1, bias1, gamma, beta, block=2000):
    N = h.shape[0]
    C = root1.shape[1]
    nb = N // block

    def body1(agg_ref, h_ref, r, b, s_ref, ps_ref, pq_ref):
        s = jnp.sum(agg_ref[...], axis=0) + _dot(h_ref[...], r[...]) + b[...]
        s = jnp.maximum(s, 0.0)
        s_ref[...] = s
        ps_ref[...] = jnp.sum(s, axis=0, keepdims=True)[None]
        pq_ref[...] = jnp.sum(s * s, axis=0, keepdims=True)[None]

    s, ps, pq = pl.pallas_call(
        body1,
        grid=(nb,),
        in_specs=[
            pl.BlockSpec((agg.shape[0], block, C), lambda i: (0, i, 0)),
            pl.BlockSpec((block, h.shape[1]), lambda i: (i, 0)),
            _full(root1), _full(bias1),
        ],
        out_specs=[
            pl.BlockSpec((block, C), lambda i: (i, 0)),
            pl.BlockSpec((1, 1, C), lambda i: (i, 0, 0)),
            pl.BlockSpec((1, 1, C), lambda i: (i, 0, 0)),
        ],
        out_shape=[
            jax.ShapeDtypeStruct((N, C), jnp.float32),
            jax.ShapeDtypeStruct((nb, 1, C), jnp.float32),
            jax.ShapeDtypeStruct((nb, 1, C), jnp.float32),
        ],
    )(agg, h, root1, bias1)

    def body2(s_ref, ps_ref, pq_ref, g, be, out_ref):
        mu = jnp.sum(ps_ref[...], axis=0) / N
        msq = jnp.sum(pq_ref[...], axis=0) / N
        var = msq - mu * mu
        out_ref[...] = ((s_ref[...] - mu) * lax.rsqrt(var + 1e-5) * g[...]
                        + be[...])

    return pl.pallas_call(
        body2,
        grid=(nb,),
        in_specs=[
            pl.BlockSpec((block, C), lambda i: (i, 0)),
            _full(ps), _full(pq), _full(gamma), _full(beta),
        ],
        out_specs=pl.BlockSpec((block, C), lambda i: (i, 0)),
        out_shape=jax.ShapeDtypeStruct((N, C), jnp.float32),
    )(s, ps, pq, gamma, beta)


def _final_tc(x1, agg2, root2, bias2, wfc_a, wfc_b, Bfc):
    N = x1.shape[0]
    NCq = wfc_a.shape[1]

    def body(x1_ref, agg_ref, r, b, wa, wb, bf, out_ref):
        x1v = x1_ref[...]
        x2 = jnp.maximum(
            jnp.sum(agg_ref[...], axis=0) + _dot(x1v, r[...]) + b[...], 0.0)
        logits = _dot(x1v, wa[...]) + _dot(x2, wb[...]) + bf[...]
        ls = logits - jnp.max(logits, axis=1, keepdims=True)
        lse = jnp.log(jnp.sum(jnp.exp(ls), axis=1, keepdims=True))
        out_ref[...] = ls - lse

    block = 2000
    C = x1.shape[1]
    return pl.pallas_call(
        body,
        grid=(N // block,),
        in_specs=[
            pl.BlockSpec((block, C), lambda i: (i, 0)),
            pl.BlockSpec((agg2.shape[0], block, NCq), lambda i: (0, i, 0)),
            _full(root2), _full(bias2), _full(wfc_a), _full(wfc_b),
            _full(Bfc),
        ],
        out_specs=pl.BlockSpec((block, NCq), lambda i: (i, 0)),
        out_shape=jax.ShapeDtypeStruct((N, NCq), jnp.float32),
    )(x1, agg2, root2, bias2, wfc_a, wfc_b, Bfc)


def kernel(x, edge_index, edge_attr, time_weights,
           Wn1, Bn1, Wn2, Bn2, We1, Be1, We2, Be2, Wt, Bt,
           c1w1, c1b1, c1w2, c1b2, c1w3, c1b3, root1, bias1,
           bn_gamma, bn_beta,
           c2w1, c2b1, c2w2, c2b2, c2w3, c2b3, root2, bias2,
           Wfc, Bfc):
    N = x.shape[0]
    DIM = Wn1.shape[1]
    CONV = root1.shape[1]
    NC2 = c2w3.shape[1]
    src = edge_index[0].astype(jnp.int32)
    dst = edge_index[1].astype(jnp.int32)
    E = src.shape[0]

    P = 8
    EP = E // P
    r2 = lambda v: v.reshape(1, -1)
    bd = lambda w: jnp.kron(jnp.eye(P, dtype=w.dtype), w)
    bt = lambda b: jnp.tile(b, P).reshape(1, -1)

    er_p = edge_attr.reshape(EP, -1)

    S = 2
    E2 = E // S
    EP2 = EP // S
    CH = 1000

    h = _node_encoder_tc(x, time_weights, Wn1, r2(Bn1), Wn2, r2(Bn2),
                         Wt, r2(Bt))

    def conv(table, w1a, w1b, b1, w2, b2, w3, b3, dout):
        hss = [_sc_gather(table, src[i * E2:(i + 1) * E2], chunk=CH)
               for i in range(S)]
        ms = [_edge_mlp_tc(hss[i].reshape(EP2, -1),
                           er_p[i * EP2:(i + 1) * EP2],
                           bd(We1), bt(Be1), bd(We2), bt(Be2),
                           bd(w1a), bd(w1b), bt(b1),
                           bd(w2), bt(b2), bd(w3), bt(b3), block=2000)
              for i in range(S)]
        zeros = jnp.zeros((N, dout), jnp.float32)
        parts = [_sc_scatter_add(ms[i].reshape(E2, dout),
                                 dst[i * E2:(i + 1) * E2], zeros,
                                 chunk=CH) for i in range(S)]
        return jnp.concatenate(parts, axis=0)

    agg = conv(h, c1w1[:DIM], c1w1[DIM:], c1b1, c1w2, c1b2, c1w3, c1b3, CONV)
    x1 = _combine_bn_tc(agg, h, root1, r2(bias1), r2(bn_gamma), r2(bn_beta))

    agg2 = conv(x1, c2w1[:CONV], c2w1[CONV:], c2b1, c2w2, c2b2, c2w3, c2b3,
                NC2)

    return _final_tc(x1, agg2, root2, r2(bias2),
                     Wfc[:CONV], Wfc[CONV:], r2(Bfc))


# speedup vs baseline: 1.2730x; 1.2730x over previous
import functools

import jax
import jax.numpy as jnp
from jax import lax
from jax.experimental import pallas as pl
from jax.experimental.pallas import tpu as pltpu
from jax.experimental.pallas import tpu_sc as plsc

_PREC = lax.Precision.HIGHEST

_NC = 2
_NS = 16
_NW = _NC * _NS


def _lrelu(v):
    return jnp.where(v > 0, v, 0.01 * v)


def _dot(a, b):
    return jnp.dot(a, b, preferred_element_type=jnp.float32, precision=_PREC)


def _bdot(a, b):
    return jnp.dot(a.astype(jnp.bfloat16), b.astype(jnp.bfloat16),
                   preferred_element_type=jnp.float32)


def _sc_gather(table, idx, chunk=5000):
    E = idx.shape[0]
    D = table.shape[1]
    per_w = E // _NW
    mesh = plsc.VectorSubcoreMesh(core_axis_name="c", subcore_axis_name="s")

    @functools.partial(
        pl.kernel,
        out_type=jax.ShapeDtypeStruct((E, D), jnp.float32),
        mesh=mesh,
        scratch_types=[
            pltpu.VMEM((chunk,), jnp.int32),
            pltpu.VMEM((chunk, D), jnp.float32),
            pltpu.SemaphoreType.DMA,
        ],
        compiler_params=pltpu.CompilerParams(use_tc_tiling_on_sc=False),
    )
    def k(table_hbm, idx_hbm, out_hbm, idx_v, rows_v, sem):
        wid = lax.axis_index("s") * _NC + lax.axis_index("c")
        base = wid * per_w

        @pl.loop(0, per_w, step=chunk)
        def _(off):
            pltpu.sync_copy(idx_hbm.at[pl.ds(base + off, chunk)], idx_v)
            pltpu.async_copy(table_hbm.at[idx_v], rows_v, sem).wait()
            pltpu.sync_copy(rows_v, out_hbm.at[pl.ds(base + off, chunk)])

    return k(table, idx)


def _sc_scatter_add(vals, idx, zeros, chunk=5000):
    E, D = vals.shape
    n_rows = zeros.shape[0]
    per_w = E // _NW
    rows_per_s = n_rows // _NS
    mesh = plsc.VectorSubcoreMesh(core_axis_name="c", subcore_axis_name="s")

    @functools.partial(
        pl.kernel,
        out_type=jax.ShapeDtypeStruct((_NC, n_rows, D), jnp.float32),
        mesh=mesh,
        scratch_types=[
            pltpu.VMEM((chunk,), jnp.int32),
            pltpu.VMEM((chunk, D), jnp.float32),
            pltpu.VMEM_SHARED((n_rows, D), jnp.float32),
        ],
        compiler_params=pltpu.CompilerParams(use_tc_tiling_on_sc=False),
    )
    def k(vals_hbm, idx_hbm, zeros_hbm, out_hbm, idx_v, val_v, acc_sh):
        cid = lax.axis_index("c")
        sid = lax.axis_index("s")
        wid = sid * _NC + cid
        row0 = sid * rows_per_s
        pltpu.sync_copy(zeros_hbm.at[pl.ds(row0, rows_per_s)],
                        acc_sh.at[pl.ds(row0, rows_per_s)])
        plsc.subcore_barrier()

        base = wid * per_w

        @pl.loop(0, per_w, step=chunk)
        def _(off):
            pltpu.sync_copy(idx_hbm.at[pl.ds(base + off, chunk)], idx_v)
            pltpu.sync_copy(vals_hbm.at[pl.ds(base + off, chunk)], val_v)
            pltpu.sync_copy(val_v, acc_sh.at[idx_v], add=True)

        plsc.subcore_barrier()
        pltpu.sync_copy(acc_sh.at[pl.ds(row0, rows_per_s)],
                        out_hbm.at[cid, pl.ds(row0, rows_per_s)])

    return k(vals, idx, zeros)


def _full(a):
    return pl.BlockSpec(a.shape, lambda i: (0,) * a.ndim)


def _node_encoder_tc(x, tw, Wn1, Bn1, Wn2, Bn2, Wt, Bt, block=2000):
    N = x.shape[0]
    DIM = Wn1.shape[1]

    def body(x_ref, tw_ref, wn1, bn1, wn2, bn2, wt, bt, out_ref):
        h1 = _lrelu(_bdot(x_ref[...], wn1[...]) + bn1[...])
        h = _bdot(h1, wn2[...]) + bn2[...]
        t = tw_ref[...] * wt[...] + bt[...]
        out_ref[...] = h * t

    return pl.pallas_call(
        body,
        grid=(N // block,),
        in_specs=[
            pl.BlockSpec((block, x.shape[1]), lambda i: (i, 0)),
            pl.BlockSpec((block, 1), lambda i: (i, 0)),
            _full(Wn1), _full(Bn1), _full(Wn2), _full(Bn2),
            _full(Wt), _full(Bt),
        ],
        out_specs=pl.BlockSpec((block, DIM), lambda i: (i, 0)),
        out_shape=jax.ShapeDtypeStruct((N, DIM), jnp.float32),
    )(x, tw, Wn1, Bn1, Wn2, Bn2, Wt, Bt)


def _edge_mlp_tc(hs, er, We1, Be1, We2, Be2, w1a, w1b, b1, w2, b2, w3, b3,
                 block=8000):
    E = hs.shape[0]
    Dh = hs.shape[1]
    De = er.shape[1]
    Dout = w3.shape[1]
    grid = (E // block,)

    def body(hs_ref, er_ref, we1, be1, we2, be2,
             w1a_r, w1b_r, b1_r, w2_r, b2_r, w3_r, b3_r, out_ref):
        ea = _bdot(_lrelu(_bdot(er_ref[...], we1[...]) + be1[...]),
                   we2[...]) + be2[...]
        mm = _lrelu(_bdot(hs_ref[...], w1a_r[...]) +
                    _bdot(ea, w1b_r[...]) + b1_r[...])
        mm = _lrelu(_bdot(mm, w2_r[...]) + b2_r[...])
        out_ref[...] = _bdot(mm, w3_r[...]) + b3_r[...]

    return pl.pallas_call(
        body,
        grid=grid,
        in_specs=[
            pl.BlockSpec((block, Dh), lambda i: (i, 0)),
            pl.BlockSpec((block, De), lambda i: (i, 0)),
            _full(We1), _full(Be1), _full(We2), _full(Be2),
            _full(w1a), _full(w1b), _full(b1), _full(w2), _full(b2),
            _full(w3), _full(b3),
        ],
        out_specs=pl.BlockSpec((block, Dout), lambda i: (i, 0)),
        out_shape=jax.ShapeDtypeStruct((E, Dout), jnp.float32),
    )(hs, er, We1, Be1, We2, Be2, w1a, w1b, b1, w2, b2, w3, b3)


def _combine_bn_tc(agg, h, root1, bias1, gamma, beta, block=2000):
    N = h.shape[0]
    C = root1.shape[1]
    nb = N // block

    def body1(agg_ref, h_ref, r, b, s_ref, ps_ref, pq_ref):
        s = agg_ref[0] + agg_ref[1] + _dot(h_ref[...], r[...]) + b[...]
        s = jnp.maximum(s, 0.0)
        s_ref[...] = s
        ps_ref[...] = jnp.sum(s, axis=0, keepdims=True)[None]
        pq_ref[...] = jnp.sum(s * s, axis=0, keepdims=True)[None]

    s, ps, pq = pl.pallas_call(
        body1,
        grid=(nb,),
        in_specs=[
            pl.BlockSpec((2, block, C), lambda i: (0, i, 0)),
            pl.BlockSpec((block, h.shape[1]), lambda i: (i, 0)),
            _full(root1), _full(bias1),
        ],
        out_specs=[
            pl.BlockSpec((block, C), lambda i: (i, 0)),
            pl.BlockSpec((1, 1, C), lambda i: (i, 0, 0)),
            pl.BlockSpec((1, 1, C), lambda i: (i, 0, 0)),
        ],
        out_shape=[
            jax.ShapeDtypeStruct((N, C), jnp.float32),
            jax.ShapeDtypeStruct((nb, 1, C), jnp.float32),
            jax.ShapeDtypeStruct((nb, 1, C), jnp.float32),
        ],
    )(agg, h, root1, bias1)

    def body2(s_ref, ps_ref, pq_ref, g, be, out_ref):
        mu = jnp.sum(ps_ref[...], axis=0) / N
        msq = jnp.sum(pq_ref[...], axis=0) / N
        var = msq - mu * mu
        out_ref[...] = ((s_ref[...] - mu) * lax.rsqrt(var + 1e-5) * g[...]
                        + be[...])

    return pl.pallas_call(
        body2,
        grid=(nb,),
        in_specs=[
            pl.BlockSpec((block, C), lambda i: (i, 0)),
            _full(ps), _full(pq), _full(gamma), _full(beta),
        ],
        out_specs=pl.BlockSpec((block, C), lambda i: (i, 0)),
        out_shape=jax.ShapeDtypeStruct((N, C), jnp.float32),
    )(s, ps, pq, gamma, beta)


def _final_tc(x1, agg2, root2, bias2, wfc_a, wfc_b, Bfc):
    N = x1.shape[0]
    NCq = wfc_a.shape[1]

    def body(x1_ref, agg_ref, r, b, wa, wb, bf, out_ref):
        x1v = x1_ref[...]
        x2 = jnp.maximum(
            agg_ref[0] + agg_ref[1] + _dot(x1v, r[...]) + b[...], 0.0)
        logits = _dot(x1v, wa[...]) + _dot(x2, wb[...]) + bf[...]
        ls = logits - jnp.max(logits, axis=1, keepdims=True)
        lse = jnp.log(jnp.sum(jnp.exp(ls), axis=1, keepdims=True))
        out_ref[...] = ls - lse

    block = 2000
    C = x1.shape[1]
    return pl.pallas_call(
        body,
        grid=(N // block,),
        in_specs=[
            pl.BlockSpec((block, C), lambda i: (i, 0)),
            pl.BlockSpec((2, block, NCq), lambda i: (0, i, 0)),
            _full(root2), _full(bias2), _full(wfc_a), _full(wfc_b),
            _full(Bfc),
        ],
        out_specs=pl.BlockSpec((block, NCq), lambda i: (i, 0)),
        out_shape=jax.ShapeDtypeStruct((N, NCq), jnp.float32),
    )(x1, agg2, root2, bias2, wfc_a, wfc_b, Bfc)


def kernel(x, edge_index, edge_attr, time_weights,
           Wn1, Bn1, Wn2, Bn2, We1, Be1, We2, Be2, Wt, Bt,
           c1w1, c1b1, c1w2, c1b2, c1w3, c1b3, root1, bias1,
           bn_gamma, bn_beta,
           c2w1, c2b1, c2w2, c2b2, c2w3, c2b3, root2, bias2,
           Wfc, Bfc):
    N = x.shape[0]
    DIM = Wn1.shape[1]
    CONV = root1.shape[1]
    NC2 = c2w3.shape[1]
    src = edge_index[0].astype(jnp.int32)
    dst = edge_index[1].astype(jnp.int32)
    E = src.shape[0]

    P = 8
    EP = E // P
    r2 = lambda v: v.reshape(1, -1)
    bd = lambda w: jnp.kron(jnp.eye(P, dtype=w.dtype), w)
    bt = lambda b: jnp.tile(b, P).reshape(1, -1)

    er_p = edge_attr.reshape(EP, -1)

    h = _node_encoder_tc(x, time_weights, Wn1, r2(Bn1), Wn2, r2(Bn2),
                         Wt, r2(Bt))

    hs = _sc_gather(h, src)
    m_p = _edge_mlp_tc(hs.reshape(EP, -1), er_p,
                       bd(We1), bt(Be1), bd(We2), bt(Be2),
                       bd(c1w1[:DIM]), bd(c1w1[DIM:]), bt(c1b1),
                       bd(c1w2), bt(c1b2), bd(c1w3), bt(c1b3), block=2000)
    agg = _sc_scatter_add(m_p.reshape(E, CONV), dst,
                          jnp.zeros((N, CONV), jnp.float32))
    x1 = _combine_bn_tc(agg, h, root1, r2(bias1), r2(bn_gamma), r2(bn_beta))

    x1s = _sc_gather(x1, src)
    m2_p = _edge_mlp_tc(x1s.reshape(EP, -1), er_p,
                        bd(We1), bt(Be1), bd(We2), bt(Be2),
                        bd(c2w1[:CONV]), bd(c2w1[CONV:]), bt(c2b1),
                        bd(c2w2), bt(c2b2), bd(c2w3), bt(c2b3), block=2000)
    agg2 = _sc_scatter_add(m2_p.reshape(E, NC2), dst,
                           jnp.zeros((N, NC2), jnp.float32))

    return _final_tc(x1, agg2, root2, r2(bias2),
                     Wfc[:CONV], Wfc[CONV:], r2(Bfc))
